# baseline (speedup 1.0000x reference)
import numpy as np

import jax
import jax.numpy as jnp
from jax.experimental import pallas as pl
from jax.experimental.pallas import tpu as pltpu
from jax.sharding import Mesh, PartitionSpec as P

B, NCAM, J = 2, 6, 23
HM = 160
W, H = 640, 512
D = 50
GRID_SIZE = 200.0
GRID_SPACING = 2.0
BAND = HM + 8


JC = 6


def _place_body(x0_ref, y0_ref, hm_ref, out_ref, pad_ref):
    i = pl.program_id(0)

    @pl.when(i == 0)
    def _():
        pad_ref[...] = jnp.zeros((BAND, W), jnp.float32)

    for k in range(JC):
        idx = i * JC + k
        x0 = x0_ref[idx]
        y0 = y0_ref[idx]
        pad_ref[0:HM, 0:HM] = hm_ref[k]
        ext = pad_ref[...]
        rolled = pltpu.roll(ext, x0, axis=1)
        r = jnp.bitwise_and(y0, 7)
        band = pltpu.roll(rolled, r, axis=0)
        base = pl.multiple_of((y0 >> 3) << 3, 8)
        out_ref[k] = jnp.zeros((H, W), out_ref.dtype)
        out_ref[k, pl.ds(base, BAND), :] = band


def _softargmax_body(vol_ref, sp_ref, pts_ref):
    v = vol_ref[0]
    sp = jnp.maximum(v, 0.0) + jnp.log1p(jnp.exp(-jnp.abs(v)))
    sp_ref[0] = sp
    mx = jnp.sum(sp, axis=(1, 2))
    my = jnp.sum(sp, axis=(0, 2))
    mz = jnp.sum(sp, axis=(0, 1))
    idx = jax.lax.broadcasted_iota(jnp.int32, (D,), 0).astype(jnp.float32)
    rn = 1.0 / jnp.sum(mx)
    x = jnp.sum(mx * idx) * rn
    y = jnp.sum(my * idx) * rn
    z = jnp.sum(mz * idx) * rn
    lane = jax.lax.broadcasted_iota(jnp.int32, (8, 128), 1)
    row = (jnp.where(lane == 0, x, 0.0) + jnp.where(lane == 1, y, 0.0)
           + jnp.where(lane == 2, z, 0.0))
    pts_ref[0] = row


def _place_shard(x0_all, y0_all, hm_flat):
    n = x0_all.shape[0]
    return pl.pallas_call(
        _place_body,
        grid_spec=pltpu.PrefetchScalarGridSpec(
            num_scalar_prefetch=2,
            grid=(n // JC,),
            in_specs=[pl.BlockSpec((JC, HM, HM),
                                   lambda i, x0r, y0r: (i, 0, 0))],
            out_specs=pl.BlockSpec((JC, H, W), lambda i, x0r, y0r: (i, 0, 0)),
            scratch_shapes=[pltpu.VMEM((BAND, W), jnp.float32)],
        ),
        out_shape=jax.ShapeDtypeStruct((n, H, W), jnp.float32),
        compiler_params=pltpu.CompilerParams(
            dimension_semantics=("arbitrary",)),
    )(x0_all, y0_all, hm_flat)


def _softargmax_shard(vol_flat):
    n = vol_flat.shape[0]
    return pl.pallas_call(
        _softargmax_body,
        grid=(n,),
        in_specs=[pl.BlockSpec((1, D, D, D), lambda i: (i, 0, 0, 0))],
        out_specs=[
            pl.BlockSpec((1, D, D, D), lambda i: (i, 0, 0, 0)),
            pl.BlockSpec((1, 8, 128), lambda i: (i, 0, 0)),
        ],
        out_shape=[
            jax.ShapeDtypeStruct((n, D, D, D), jnp.float32),
            jax.ShapeDtypeStruct((n, 8, 128), jnp.float32),
        ],
        compiler_params=pltpu.CompilerParams(
            dimension_semantics=("arbitrary",)),
    )(vol_flat)


def kernel(heatmaps_batch, heatmap_vol, center3D, centerHM):
    hm_flat = heatmaps_batch.reshape(B * NCAM * J, HM, HM)
    x0 = (centerHM[..., 0] // 2 - HM // 2).reshape(-1).astype(jnp.int32)
    y0 = (centerHM[..., 1] // 2 - HM // 2).reshape(-1).astype(jnp.int32)
    x0_all = jnp.repeat(x0, J)
    y0_all = jnp.repeat(y0, J)
    vol_flat = heatmap_vol.reshape(B * J, D, D, D)

    devs = jax.devices()
    if len(devs) >= 2:
        mesh = Mesh(np.array(devs[:2]), ("c",))
        placed = jax.shard_map(
            _place_shard, mesh=mesh,
            in_specs=(P("c"), P("c"), P("c")), out_specs=P("c"),
            check_vma=False,
        )(x0_all, y0_all, hm_flat)
        sp, pts = jax.shard_map(
            _softargmax_shard, mesh=mesh,
            in_specs=(P("c"),), out_specs=(P("c"), P("c")),
            check_vma=False,
        )(vol_flat)
    else:
        placed = _place_shard(x0_all, y0_all, hm_flat)
        sp, pts = _softargmax_shard(vol_flat)

    heatmaps_padded = placed.reshape(B, NCAM, J, H, W)
    heatmap_final = sp.reshape(B, J, D, D, D)
    raw = pts[:, 0, :3].reshape(B, J, 3)
    points3D = raw * (GRID_SPACING * 2.0) - GRID_SIZE / GRID_SPACING + center3D
    return (heatmap_final, heatmaps_padded, points3D)


# speedup vs baseline: 1.0085x; 1.0085x over previous
import jax
import jax.numpy as jnp
from jax.experimental import pallas as pl
from jax.experimental.pallas import tpu as pltpu

B, NCAM, J = 2, 6, 23
HM = 160
W, H = 640, 512
D = 50
GRID_SIZE = 200.0
GRID_SPACING = 2.0
BAND = HM + 8
JC = 6


def _place_body(x0_ref, y0_ref, hm_ref, out_ref, pad_ref):
    i = pl.program_id(0)

    @pl.when(i == 0)
    def _():
        pad_ref[...] = jnp.zeros((BAND, W), jnp.float32)

    for k in range(JC):
        idx = i * JC + k
        x0 = x0_ref[idx]
        y0 = y0_ref[idx]
        pad_ref[0:HM, 0:HM] = hm_ref[k]
        ext = pad_ref[...]
        rolled = pltpu.roll(ext, x0, axis=1)
        r = jnp.bitwise_and(y0, 7)
        band = pltpu.roll(rolled, r, axis=0)
        base = pl.multiple_of((y0 >> 3) << 3, 8)
        out_ref[k] = jnp.zeros((H, W), out_ref.dtype)
        out_ref[k, pl.ds(base, BAND), :] = band


def _softargmax_body(vol_ref, sp_ref, pts_ref):
    v = vol_ref[0, 0]
    sp = jnp.maximum(v, 0.0) + jnp.log1p(jnp.exp(-jnp.abs(v)))
    sp_ref[0, 0] = sp
    mx = jnp.sum(sp, axis=(1, 2))
    my = jnp.sum(sp, axis=(0, 2))
    mz = jnp.sum(sp, axis=(0, 1))
    idx = jax.lax.broadcasted_iota(jnp.int32, (D,), 0).astype(jnp.float32)
    rn = 1.0 / jnp.sum(mx)
    x = jnp.sum(mx * idx) * rn
    y = jnp.sum(my * idx) * rn
    z = jnp.sum(mz * idx) * rn
    lane = jax.lax.broadcasted_iota(jnp.int32, (8, 128), 1)
    row = (jnp.where(lane == 0, x, 0.0) + jnp.where(lane == 1, y, 0.0)
           + jnp.where(lane == 2, z, 0.0))
    pts_ref[0, 0] = row


def kernel(heatmaps_batch, heatmap_vol, center3D, centerHM):
    hm_flat = heatmaps_batch.reshape(B * NCAM * J, HM, HM)
    x0 = (centerHM[..., 0] // 2 - HM // 2).reshape(-1).astype(jnp.int32)
    y0 = (centerHM[..., 1] // 2 - HM // 2).reshape(-1).astype(jnp.int32)
    x0_all = jnp.repeat(x0, J)
    y0_all = jnp.repeat(y0, J)

    placed = pl.pallas_call(
        _place_body,
        grid_spec=pltpu.PrefetchScalarGridSpec(
            num_scalar_prefetch=2,
            grid=(B * NCAM * J // JC,),
            in_specs=[pl.BlockSpec((JC, HM, HM),
                                   lambda i, x0r, y0r: (i, 0, 0))],
            out_specs=pl.BlockSpec((JC, H, W), lambda i, x0r, y0r: (i, 0, 0)),
            scratch_shapes=[pltpu.VMEM((BAND, W), jnp.float32)],
        ),
        out_shape=jax.ShapeDtypeStruct((B * NCAM * J, H, W), jnp.float32),
        compiler_params=pltpu.CompilerParams(
            dimension_semantics=("arbitrary",)),
    )(x0_all, y0_all, hm_flat)
    heatmaps_padded = placed.reshape(B, NCAM, J, H, W)

    heatmap_final, pts = pl.pallas_call(
        _softargmax_body,
        grid=(B * J,),
        in_specs=[pl.BlockSpec((1, 1, D, D, D),
                               lambda i: (i // J, i % J, 0, 0, 0))],
        out_specs=[
            pl.BlockSpec((1, 1, D, D, D), lambda i: (i // J, i % J, 0, 0, 0)),
            pl.BlockSpec((1, 1, 8, 128), lambda i: (i // J, i % J, 0, 0)),
        ],
        out_shape=[
            jax.ShapeDtypeStruct((B, J, D, D, D), jnp.float32),
            jax.ShapeDtypeStruct((B, J, 8, 128), jnp.float32),
        ],
        compiler_params=pltpu.CompilerParams(
            dimension_semantics=("arbitrary",)),
    )(heatmap_vol)
    raw = pts[:, :, 0, :3]
    points3D = raw * (GRID_SPACING * 2.0) - GRID_SIZE / GRID_SPACING + center3D
    return (heatmap_final, heatmaps_padded, points3D)


# speedup vs baseline: 1.1656x; 1.1557x over previous
import jax
import jax.numpy as jnp
from jax.experimental import pallas as pl
from jax.experimental.pallas import tpu as pltpu

B, NCAM, J = 2, 6, 23
HM = 160
W, H = 640, 512
D = 50
GRID_SIZE = 200.0
GRID_SPACING = 2.0
BAND = HM + 8
JC = 6


def _place_body(x0_ref, y0_ref, hm_ref, out_ref, pad_ref):
    i = pl.program_id(0)

    @pl.when(i == 0)
    def _():
        pad_ref[...] = jnp.zeros((BAND, W), jnp.float32)

    for k in range(JC):
        idx = i * JC + k
        x0 = x0_ref[idx]
        y0 = y0_ref[idx]
        pad_ref[0:HM, 0:HM] = hm_ref[k]
        ext = pad_ref[...]
        rolled = pltpu.roll(ext, x0, axis=1)
        r = jnp.bitwise_and(y0, 7)
        band = pltpu.roll(rolled, r, axis=0)
        base = pl.multiple_of((y0 >> 3) << 3, 8)
        out_ref[k] = jnp.zeros((H, W), out_ref.dtype)
        out_ref[k, pl.ds(base, BAND), :] = band


def _softargmax_body(vol_ref, sp_ref, pts_ref):
    v = vol_ref[0]
    sp = jnp.maximum(v, 0.0) + jnp.log1p(jnp.exp(-jnp.abs(v)))
    sp_ref[0] = sp
    mx = jnp.sum(sp, axis=(1, 2))
    my = jnp.sum(sp, axis=(0, 2))
    mz = jnp.sum(sp, axis=(0, 1))
    idx = jax.lax.broadcasted_iota(jnp.int32, (D,), 0).astype(jnp.float32)
    rn = 1.0 / jnp.sum(mx)
    x = jnp.sum(mx * idx) * rn
    y = jnp.sum(my * idx) * rn
    z = jnp.sum(mz * idx) * rn
    lane = jax.lax.broadcasted_iota(jnp.int32, (8, 128), 1)
    row = (jnp.where(lane == 0, x, 0.0) + jnp.where(lane == 1, y, 0.0)
           + jnp.where(lane == 2, z, 0.0))
    pts_ref[0] = row


def kernel(heatmaps_batch, heatmap_vol, center3D, centerHM):
    hm_flat = heatmaps_batch.reshape(B * NCAM * J, HM, HM)
    x0 = (centerHM[..., 0] // 2 - HM // 2).reshape(-1).astype(jnp.int32)
    y0 = (centerHM[..., 1] // 2 - HM // 2).reshape(-1).astype(jnp.int32)
    x0_all = jnp.repeat(x0, J)
    y0_all = jnp.repeat(y0, J)

    placed = pl.pallas_call(
        _place_body,
        grid_spec=pltpu.PrefetchScalarGridSpec(
            num_scalar_prefetch=2,
            grid=(B * NCAM * J // JC,),
            in_specs=[pl.BlockSpec((JC, HM, HM),
                                   lambda i, x0r, y0r: (i, 0, 0))],
            out_specs=pl.BlockSpec((JC, H, W), lambda i, x0r, y0r: (i, 0, 0)),
            scratch_shapes=[pltpu.VMEM((BAND, W), jnp.float32)],
        ),
        out_shape=jax.ShapeDtypeStruct((B * NCAM * J, H, W), jnp.float32),
        compiler_params=pltpu.CompilerParams(
            dimension_semantics=("arbitrary",)),
    )(x0_all, y0_all, hm_flat)
    heatmaps_padded = placed.reshape(B, NCAM, J, H, W)

    vol_flat = heatmap_vol.reshape(B * J, D, D, D)
    sp, pts = pl.pallas_call(
        _softargmax_body,
        grid=(B * J,),
        in_specs=[pl.BlockSpec((1, D, D, D), lambda i: (i, 0, 0, 0))],
        out_specs=[
            pl.BlockSpec((1, D, D, D), lambda i: (i, 0, 0, 0)),
            pl.BlockSpec((1, 8, 128), lambda i: (i, 0, 0)),
        ],
        out_shape=[
            jax.ShapeDtypeStruct((B * J, D, D, D), jnp.float32),
            jax.ShapeDtypeStruct((B * J, 8, 128), jnp.float32),
        ],
        compiler_params=pltpu.CompilerParams(
            dimension_semantics=("arbitrary",)),
    )(vol_flat)
    heatmap_final = sp.reshape(B, J, D, D, D)
    raw = pts[:, 0, :3].reshape(B, J, 3)
    points3D = raw * (GRID_SPACING * 2.0) - GRID_SIZE / GRID_SPACING + center3D
    return (heatmap_final, heatmaps_padded, points3D)


# speedup vs baseline: 1.2126x; 1.0404x over previous
import jax
import jax.numpy as jnp
from jax.experimental import pallas as pl
from jax.experimental.pallas import tpu as pltpu

B, NCAM, J = 2, 6, 23
HM = 160
W, H = 640, 512
D = 50
GRID_SIZE = 200.0
GRID_SPACING = 2.0
BAND = HM + 8
JC = 6


def _place_body(x0_ref, y0_ref, hm_ref, out_ref, pad_ref):
    i = pl.program_id(0)

    @pl.when(i == 0)
    def _():
        pad_ref[...] = jnp.zeros((BAND, W), jnp.float32)

    for k in range(JC):
        idx = i * JC + k
        x0 = x0_ref[idx]
        y0 = y0_ref[idx]
        pad_ref[0:HM, 0:HM] = hm_ref[k]
        ext = pad_ref[...]
        rolled = pltpu.roll(ext, x0, axis=1)
        r = jnp.bitwise_and(y0, 7)
        band = pltpu.roll(rolled, r, axis=0)
        base = pl.multiple_of((y0 >> 3) << 3, 8)
        out_ref[k] = jnp.zeros((H, W), out_ref.dtype)
        out_ref[k, pl.ds(base, BAND), :] = band


JV = 2


def _softargmax_body(vol_ref, sp_ref, pts_ref):
    lane = jax.lax.broadcasted_iota(jnp.int32, (8, 128), 1)
    idx = jax.lax.broadcasted_iota(jnp.int32, (D,), 0).astype(jnp.float32)
    for k in range(JV):
        v = vol_ref[k]
        sp = jnp.maximum(v, 0.0) + jnp.log1p(jnp.exp(-jnp.abs(v)))
        sp_ref[k] = sp
        s0 = jnp.sum(sp, axis=0)
        my = jnp.sum(s0, axis=1)
        mz = jnp.sum(s0, axis=0)
        mx = jnp.sum(sp, axis=(1, 2))
        rn = 1.0 / jnp.sum(mz)
        x = jnp.sum(mx * idx) * rn
        y = jnp.sum(my * idx) * rn
        z = jnp.sum(mz * idx) * rn
        row = (jnp.where(lane == 0, x, 0.0) + jnp.where(lane == 1, y, 0.0)
               + jnp.where(lane == 2, z, 0.0))
        pts_ref[k] = row


def kernel(heatmaps_batch, heatmap_vol, center3D, centerHM):
    hm_flat = heatmaps_batch.reshape(B * NCAM * J, HM, HM)
    x0 = (centerHM[..., 0] // 2 - HM // 2).reshape(-1).astype(jnp.int32)
    y0 = (centerHM[..., 1] // 2 - HM // 2).reshape(-1).astype(jnp.int32)
    x0_all = jnp.repeat(x0, J)
    y0_all = jnp.repeat(y0, J)


    vol_flat = heatmap_vol.reshape(B * J, D, D, D)
    sp, pts = pl.pallas_call(
        _softargmax_body,
        grid=(B * J // JV,),
        in_specs=[pl.BlockSpec((JV, D, D, D), lambda i: (i, 0, 0, 0))],
        out_specs=[
            pl.BlockSpec((JV, D, D, D), lambda i: (i, 0, 0, 0)),
            pl.BlockSpec((JV, 8, 128), lambda i: (i, 0, 0)),
        ],
        out_shape=[
            jax.ShapeDtypeStruct((B * J, D, D, D), jnp.float32),
            jax.ShapeDtypeStruct((B * J, 8, 128), jnp.float32),
        ],
        compiler_params=pltpu.CompilerParams(
            dimension_semantics=("arbitrary",)),
    )(vol_flat)
    placed = pl.pallas_call(
        _place_body,
        grid_spec=pltpu.PrefetchScalarGridSpec(
            num_scalar_prefetch=2,
            grid=(B * NCAM * J // JC,),
            in_specs=[pl.BlockSpec((JC, HM, HM),
                                   lambda i, x0r, y0r: (i, 0, 0))],
            out_specs=pl.BlockSpec((JC, H, W), lambda i, x0r, y0r: (i, 0, 0)),
            scratch_shapes=[pltpu.VMEM((BAND, W), jnp.float32)],
        ),
        out_shape=jax.ShapeDtypeStruct((B * NCAM * J, H, W), jnp.float32),
        compiler_params=pltpu.CompilerParams(
            dimension_semantics=("arbitrary",)),
    )(x0_all, y0_all, hm_flat)
    heatmaps_padded = placed.reshape(B, NCAM, J, H, W)
    heatmap_final = sp.reshape(B, J, D, D, D)
    raw = pts[:, 0, :3].reshape(B, J, 3)
    points3D = raw * (GRID_SPACING * 2.0) - GRID_SIZE / GRID_SPACING + center3D
    return (heatmap_final, heatmaps_padded, points3D)
